# baseline (speedup 1.0000x reference)
import functools

import jax
import jax.numpy as jnp
from jax import lax
from jax.experimental import pallas as pl
from jax.experimental.pallas import tpu as pltpu
from jax.experimental.pallas import tpu_sc as plsc

_B, _S, _C, _M = 16, 512, 256, 2048
_NW = 32
_FPW = _M * _B // _NW
_CH = 128
_NCH = _FPW // _CH
_L = 16


def _sc_body(xpad_hbm, dur_hbm, ml_hbm, out_hbm, mask_hbm,
             dur_v, cs_v, ml_v, idx_v, msk_v, bufa, bufb,
             ga, gb, wa, wb):
    wid = lax.axis_index("s") * 2 + lax.axis_index("c")
    b = wid // 2
    fb = (wid % 2) * _FPW

    pltpu.sync_copy(ml_hbm, ml_v)
    pltpu.sync_copy(dur_hbm.at[b], dur_v)
    mlv = ml_v[...]

    carry = jnp.int32(0)
    for j in range(_S // _L):
        d_i = dur_v[pl.ds(j * _L, _L)].astype(jnp.int32)
        cc = jnp.minimum(plsc.cumsum(d_i) + carry, mlv)
        cs_v[pl.ds(j * _L, _L)] = cc
        carry = carry + jnp.sum(d_i)
    total_v = jnp.minimum(jnp.full((_L,), carry, jnp.int32), mlv)

    lane = lax.iota(jnp.int32, _L)
    zrow = jnp.int32(b * (_S + 1) + _S)

    def _index_chunk(c):
        def body(v, _):
            t = fb + c * _CH + v * _L + lane
            lo = jnp.zeros((_L,), jnp.int32)
            hi = jnp.full((_L,), _S, jnp.int32)
            for _ in range(10):
                mid = (lo + hi) >> 1
                le = plsc.load_gather(cs_v, [mid]) <= t
                lo = jnp.where(le, mid + 1, lo)
                hi = jnp.where(le, hi, mid)
            valid = t < total_v
            g = jnp.where(valid, b * (_S + 1) + lo, zrow)
            idx_v[pl.ds(c * _CH + v * _L, _L)] = g
            msk_v[pl.ds(c * _CH + v * _L, _L)] = (~valid).astype(jnp.int32)
            return 0

        lax.fori_loop(0, _CH // _L, body, 0)

    bufs, gsems, wsems = [bufa, bufb], [ga, gb], [wa, wb]
    gh = [None] * _NCH
    wh = [None] * _NCH
    obase = wid * _FPW
    for c in range(_NCH):
        _index_chunk(c)
        if c >= 2:
            wh[c - 2].wait()
        gh[c] = pltpu.async_copy(
            xpad_hbm.at[idx_v.at[pl.ds(c * _CH, _CH)]], bufs[c % 2],
            gsems[c % 2])
        if c >= 1:
            gh[c - 1].wait()
            wh[c - 1] = pltpu.async_copy(
                bufs[(c - 1) % 2], out_hbm.at[pl.ds(obase + (c - 1) * _CH, _CH)],
                wsems[(c - 1) % 2])
    gh[_NCH - 1].wait()
    wh[_NCH - 1] = pltpu.async_copy(
        bufs[(_NCH - 1) % 2],
        out_hbm.at[pl.ds(obase + (_NCH - 1) * _CH, _CH)],
        wsems[(_NCH - 1) % 2])
    wh[_NCH - 2].wait()
    wh[_NCH - 1].wait()

    pltpu.sync_copy(msk_v, mask_hbm.at[pl.ds(obase, _FPW)])


_sc_expand = functools.partial(
    pl.kernel,
    mesh=plsc.VectorSubcoreMesh(core_axis_name="c", subcore_axis_name="s"),
    out_type=[
        jax.ShapeDtypeStruct((_B * _M, _C), jnp.float32),
        jax.ShapeDtypeStruct((_B * _M,), jnp.int32),
    ],
    scratch_types=[
        pltpu.VMEM((_S,), jnp.float32),
        pltpu.VMEM((_S,), jnp.int32),
        pltpu.VMEM((_L,), jnp.int32),
        pltpu.VMEM((_FPW,), jnp.int32),
        pltpu.VMEM((_FPW,), jnp.int32),
        pltpu.VMEM((_CH, _C), jnp.float32),
        pltpu.VMEM((_CH, _C), jnp.float32),
        pltpu.SemaphoreType.DMA,
        pltpu.SemaphoreType.DMA,
        pltpu.SemaphoreType.DMA,
        pltpu.SemaphoreType.DMA,
    ],
    compiler_params=pltpu.CompilerParams(needs_layout_passes=False),
)(_sc_body)


def kernel(x, durations, max_length):
    B, S, C = x.shape
    xpad = jnp.concatenate(
        [x, jnp.zeros((B, 1, C), x.dtype)], axis=1).reshape(B * (S + 1), C)
    d = jnp.round(durations)
    ml = jnp.full((_L,), max_length, jnp.int32)
    out, mask_i = _sc_expand(xpad, d, ml)
    expanded = out.reshape(B, _M, C)
    mel_masks = mask_i.reshape(B, _M).astype(bool)
    return expanded, mel_masks
